# baseline (speedup 1.0000x reference)
import functools

import jax
import jax.numpy as jnp
from jax import lax
from jax.experimental import pallas as pl
from jax.experimental.pallas import tpu as pltpu
from jax.experimental.pallas import tpu_sc as plsc

N = 10000
E = 320000
C = 128
NUM_CONV = 4
NUM_GRAPHS = 64
H1 = 64
NUM_CLASSES = 10

NC = 2
NS = 16
NW = NC * NS
EPW = E // NW
CH = 50
NCH = EPW // CH
NPH = 5
PCH = NCH // NPH
NBUF = 5
QUAD = PCH // NBUF
NP = 10112
RPT = NP // NS

CHD = 125
NCHD = EPW // CHD
NP1 = 10240
RPT1 = NP1 // NS

_sc_mesh = plsc.VectorSubcoreMesh(core_axis_name="c", subcore_axis_name="s")


@functools.partial(
    pl.kernel,
    out_type=jax.ShapeDtypeStruct((NC, NP1), jnp.float32),
    mesh=_sc_mesh,
    scratch_types=[
        pltpu.VMEM((NCHD, CHD), jnp.int32),
        pltpu.VMEM((128,), jnp.float32),
        pltpu.VMEM((RPT1,), jnp.float32),
        pltpu.MemorySpace.VMEM_SHARED((NP1,), jnp.float32),
    ],
)
def _sc_degree(edges_hbm, out_hbm, idx_v, ones_v, zer_v, acc_sh):
    c = lax.axis_index("c")
    s = lax.axis_index("s")
    wid = s * NC + c
    pltpu.sync_copy(edges_hbm.at[0, wid], idx_v)
    for i in range(128 // 16):
        ones_v[pl.ds(i * 16, 16)] = jnp.ones((16,), jnp.float32)
    for i in range(RPT1 // 16):
        zer_v[pl.ds(i * 16, 16)] = jnp.zeros((16,), jnp.float32)
    pltpu.sync_copy(zer_v, acc_sh.at[pl.ds(s * RPT1, RPT1)])
    plsc.subcore_barrier()

    def body(j, _):
        pltpu.sync_copy(ones_v.at[pl.ds(0, CHD)], acc_sh.at[idx_v.at[j]],
                        add=True)
        return 0

    lax.fori_loop(0, NCHD, body, 0)
    plsc.subcore_barrier()
    pltpu.sync_copy(acc_sh.at[pl.ds(s * RPT1, RPT1)],
                    out_hbm.at[c, pl.ds(s * RPT1, RPT1)])


@functools.partial(
    pl.kernel,
    out_type=jax.ShapeDtypeStruct((NC, NP, C), jnp.float32),
    mesh=_sc_mesh,
    scratch_types=[
        pltpu.VMEM((PCH, CH), jnp.int32),
        pltpu.VMEM((PCH, CH), jnp.int32),
        pltpu.VMEM((CH, C), jnp.float32),
        pltpu.VMEM((CH, C), jnp.float32),
        pltpu.VMEM((CH, C), jnp.float32),
        pltpu.VMEM((CH, C), jnp.float32),
        pltpu.VMEM((CH, C), jnp.float32),
        pltpu.MemorySpace.VMEM_SHARED((NP, C), jnp.float32),
        pltpu.SemaphoreType.DMA,
        pltpu.SemaphoreType.DMA,
        pltpu.SemaphoreType.DMA,
        pltpu.SemaphoreType.DMA,
        pltpu.SemaphoreType.DMA,
        pltpu.SemaphoreType.DMA,
    ],
)
def _sc_conv(g_hbm, edges_hbm, zeros_hbm, out_hbm,
             src_v, dst_v, rows_a, rows_b, rows_c, rows_d, rows_e, acc_sh,
             gsem_a, gsem_b, gsem_c, gsem_d, gsem_e, zsem):
    c = lax.axis_index("c")
    s = lax.axis_index("s")
    wid = s * NC + c
    zero_cp = pltpu.async_copy(zeros_hbm, acc_sh.at[pl.ds(s * RPT, RPT)],
                               zsem)

    def gather(j, buf, sem):
        return pltpu.async_copy(g_hbm.at[src_v.at[j]], buf, sem)

    def wait_gather(j, buf, sem):
        pltpu.make_async_copy(g_hbm.at[src_v.at[j]], buf, sem).wait()

    def scatter(j, buf):
        pltpu.sync_copy(buf, acc_sh.at[dst_v.at[j]], add=True)

    bufs = [(rows_a, gsem_a), (rows_b, gsem_b), (rows_c, gsem_c),
            (rows_d, gsem_d), (rows_e, gsem_e)]
    for p in range(NPH):
        pltpu.sync_copy(edges_hbm.at[0, wid, p], src_v)
        pltpu.sync_copy(edges_hbm.at[1, wid, p], dst_v)
        for k, (buf, sem) in enumerate(bufs):
            gather(k, buf, sem)
        if p == 0:
            zero_cp.wait()
            plsc.subcore_barrier()

        def quad(j4, _):
            for k, (buf, sem) in enumerate(bufs):
                wait_gather(j4 + k, buf, sem)
                scatter(j4 + k, buf)

                @pl.when(j4 + k + NBUF < PCH)
                def _():
                    gather(j4 + k + NBUF, buf, sem)
            return 0

        lax.fori_loop(0, QUAD, lambda t, u: quad(t * NBUF, u), 0)
    plsc.subcore_barrier()
    pltpu.sync_copy(acc_sh.at[pl.ds(s * RPT, RPT)],
                    out_hbm.at[c, pl.ds(s * RPT, RPT)])


def _tc_pre_body(deg_ref, x_ref, dinv_ref, g_ref):
    deg = deg_ref[...][:, 0:1] + deg_ref[...][:, 1:2]
    dinv = jnp.where(deg > 0.0, lax.rsqrt(jnp.maximum(deg, 1e-12)), 0.0)
    dinv_ref[...] = dinv
    g_ref[...] = jnp.broadcast_to(dinv, (deg.shape[0], C)) * x_ref[...]


_RB = 1000

_tc_pre = pl.pallas_call(
    _tc_pre_body,
    grid=(N // _RB,),
    in_specs=[
        pl.BlockSpec((_RB, 2), lambda i: (i, 0)),
        pl.BlockSpec((_RB, C), lambda i: (i, 0)),
    ],
    out_specs=[
        pl.BlockSpec((_RB, 1), lambda i: (i, 0)),
        pl.BlockSpec((_RB, C), lambda i: (i, 0)),
    ],
    out_shape=[
        jax.ShapeDtypeStruct((N, 1), jnp.float32),
        jax.ShapeDtypeStruct((N, C), jnp.float32),
    ],
)


def _tc_layer_body(h_ref, a0_ref, a1_ref, dinv_ref, w0_ref, w1_ref, b_ref,
                   hn_ref, gn_ref):
    dinvb = jnp.broadcast_to(dinv_ref[...], (_RB, C))
    m = dinvb * (a0_ref[...] + a1_ref[...])
    out = (jnp.dot(h_ref[...], w0_ref[...], preferred_element_type=jnp.float32)
           - jnp.dot(m, w1_ref[...], preferred_element_type=jnp.float32)
           + b_ref[0:1, :])
    hn = jnp.maximum(out, 0.0)
    hn_ref[...] = hn
    gn_ref[...] = dinvb * hn


_tc_layer = pl.pallas_call(
    _tc_layer_body,
    grid=(N // _RB,),
    in_specs=[
        pl.BlockSpec((_RB, C), lambda i: (i, 0)),
        pl.BlockSpec((_RB, C), lambda i: (i, 0)),
        pl.BlockSpec((_RB, C), lambda i: (i, 0)),
        pl.BlockSpec((_RB, 1), lambda i: (i, 0)),
        pl.BlockSpec((C, C), lambda i: (0, 0)),
        pl.BlockSpec((C, C), lambda i: (0, 0)),
        pl.BlockSpec((8, C), lambda i: (0, 0)),
    ],
    out_specs=[
        pl.BlockSpec((_RB, C), lambda i: (i, 0)),
        pl.BlockSpec((_RB, C), lambda i: (i, 0)),
    ],
    out_shape=[
        jax.ShapeDtypeStruct((N, C), jnp.float32),
        jax.ShapeDtypeStruct((N, C), jnp.float32),
    ],
)


def _tc_last_body(h_ref, a0_ref, a1_ref, dinv_ref, w0_ref, w1_ref, b_ref,
                  batch_ref, lw1_ref, lb1_ref, lw2_ref, lb2_ref,
                  out_ref, pooled_acc, count_acc):
    i = pl.program_id(0)

    @pl.when(i == 0)
    def _():
        pooled_acc[...] = jnp.zeros_like(pooled_acc)
        count_acc[...] = jnp.zeros_like(count_acc)

    m = jnp.broadcast_to(dinv_ref[...], (_RB, C)) * (a0_ref[...] + a1_ref[...])
    hn = jnp.maximum(
        jnp.dot(h_ref[...], w0_ref[...], preferred_element_type=jnp.float32)
        - jnp.dot(m, w1_ref[...], preferred_element_type=jnp.float32)
        + b_ref[0:1, :], 0.0)

    b = batch_ref[0]
    seg = jax.lax.broadcasted_iota(jnp.int32, (NUM_GRAPHS, _RB), 0)
    onehot = (seg == jnp.broadcast_to(b, (NUM_GRAPHS, _RB))).astype(jnp.float32)
    pooled_acc[...] += jnp.dot(onehot, hn, preferred_element_type=jnp.float32)
    count_acc[...] += jnp.broadcast_to(
        jnp.sum(onehot, axis=1, keepdims=True), (NUM_GRAPHS, C))

    @pl.when(i == (N // _RB) - 1)
    def _():
        pooled = pooled_acc[...] / jnp.maximum(count_acc[...], 1.0)
        h2 = jnp.maximum(
            jnp.dot(pooled, lw1_ref[...], preferred_element_type=jnp.float32)
            + lb1_ref[0:1, :], 0.0)
        out_ref[...] = (jnp.dot(h2, lw2_ref[...],
                                preferred_element_type=jnp.float32)
                        + lb2_ref[0:1, :])


_tc_last = pl.pallas_call(
    _tc_last_body,
    grid=(N // _RB,),
    in_specs=[
        pl.BlockSpec((_RB, C), lambda i: (i, 0)),
        pl.BlockSpec((_RB, C), lambda i: (i, 0)),
        pl.BlockSpec((_RB, C), lambda i: (i, 0)),
        pl.BlockSpec((_RB, 1), lambda i: (i, 0)),
        pl.BlockSpec((C, C), lambda i: (0, 0)),
        pl.BlockSpec((C, C), lambda i: (0, 0)),
        pl.BlockSpec((8, C), lambda i: (0, 0)),
        pl.BlockSpec((1, 1, _RB), lambda i: (i, 0, 0)),
        pl.BlockSpec((C, H1), lambda i: (0, 0)),
        pl.BlockSpec((8, H1), lambda i: (0, 0)),
        pl.BlockSpec((H1, C), lambda i: (0, 0)),
        pl.BlockSpec((8, C), lambda i: (0, 0)),
    ],
    out_specs=pl.BlockSpec((NUM_GRAPHS, C), lambda i: (0, 0)),
    out_shape=jax.ShapeDtypeStruct((NUM_GRAPHS, C), jnp.float32),
    scratch_shapes=[
        pltpu.VMEM((NUM_GRAPHS, C), jnp.float32),
        pltpu.VMEM((NUM_GRAPHS, C), jnp.float32),
    ],
)


def kernel(x, edge_index, batch, conv_W, conv_b, lin1_W, lin1_b, lin2_W,
           lin2_b):
    edges_deg = edge_index.reshape(2, NW, NCHD, CHD)
    edges_conv = edge_index.reshape(2, NW, NPH, PCH, CH)
    zeros_rows = jnp.zeros((RPT, C), jnp.float32)

    deg2 = _sc_degree(edges_deg)
    degT = jnp.transpose(deg2[:, :N])
    dinv, g = _tc_pre(degT, x)

    h = x
    for i in range(NUM_CONV - 1):
        bb = jnp.broadcast_to(conv_b[i], (8, C))
        acc = _sc_conv(g, edges_conv, zeros_rows)
        h, g = _tc_layer(h, acc[0], acc[1], dinv,
                         conv_W[i, 0], conv_W[i, 1], bb)

    batch3 = batch.astype(jnp.int32).reshape(N // _RB, 1, _RB)
    b1b = jnp.broadcast_to(lin1_b, (8, H1))
    w2p = jnp.zeros((H1, C), jnp.float32).at[:, :NUM_CLASSES].set(lin2_W)
    b2p = jnp.broadcast_to(
        jnp.zeros((C,), jnp.float32).at[:NUM_CLASSES].set(lin2_b), (8, C))
    acc = _sc_conv(g, edges_conv, zeros_rows)
    bb = jnp.broadcast_to(conv_b[NUM_CONV - 1], (8, C))
    out = _tc_last(h, acc[0], acc[1], dinv,
                   conv_W[NUM_CONV - 1, 0], conv_W[NUM_CONV - 1, 1], bb,
                   batch3, lin1_W, b1b, w2p, b2p)
    return out[:, :NUM_CLASSES]


# speedup vs baseline: 1.0038x; 1.0038x over previous
import functools

import jax
import jax.numpy as jnp
from jax import lax
from jax.experimental import pallas as pl
from jax.experimental.pallas import tpu as pltpu
from jax.experimental.pallas import tpu_sc as plsc

N = 10000
E = 320000
C = 128
NUM_CONV = 4
NUM_GRAPHS = 64
H1 = 64
NUM_CLASSES = 10

NC = 2
NS = 16
NW = NC * NS
EPW = E // NW
CH = 50
NCH = EPW // CH
NPH = 5
PCH = NCH // NPH
NBUF = 5
QUAD = PCH // NBUF
NP = 10112
RPT = NP // NS

CHD = 80
NCHD = EPW // CHD
NP1 = 10240
RPT1 = NP1 // NS

_sc_mesh = plsc.VectorSubcoreMesh(core_axis_name="c", subcore_axis_name="s")


@functools.partial(
    pl.kernel,
    out_type=jax.ShapeDtypeStruct((NC, NP1), jnp.float32),
    mesh=_sc_mesh,
    scratch_types=[
        pltpu.VMEM((NCHD, CHD), jnp.int32),
        pltpu.VMEM((CHD,), jnp.float32),
        pltpu.VMEM((RPT1,), jnp.float32),
        pltpu.MemorySpace.VMEM_SHARED((NP1,), jnp.float32),
    ],
)
def _sc_degree(edges_hbm, out_hbm, idx_v, ones_v, zer_v, acc_sh):
    c = lax.axis_index("c")
    s = lax.axis_index("s")
    wid = s * NC + c
    pltpu.sync_copy(edges_hbm.at[0, wid], idx_v)
    for i in range(CHD // 16):
        ones_v[pl.ds(i * 16, 16)] = jnp.ones((16,), jnp.float32)
    for i in range(RPT1 // 16):
        zer_v[pl.ds(i * 16, 16)] = jnp.zeros((16,), jnp.float32)
    pltpu.sync_copy(zer_v, acc_sh.at[pl.ds(s * RPT1, RPT1)])
    plsc.subcore_barrier()

    def body(j, _):
        pltpu.sync_copy(ones_v, acc_sh.at[idx_v.at[j]], add=True)
        return 0

    lax.fori_loop(0, NCHD, body, 0)
    plsc.subcore_barrier()
    pltpu.sync_copy(acc_sh.at[pl.ds(s * RPT1, RPT1)],
                    out_hbm.at[c, pl.ds(s * RPT1, RPT1)])


@functools.partial(
    pl.kernel,
    out_type=jax.ShapeDtypeStruct((NC, NP, C), jnp.float32),
    mesh=_sc_mesh,
    scratch_types=[
        pltpu.VMEM((PCH, CH), jnp.int32),
        pltpu.VMEM((PCH, CH), jnp.int32),
        pltpu.VMEM((CH, C), jnp.float32),
        pltpu.VMEM((CH, C), jnp.float32),
        pltpu.VMEM((CH, C), jnp.float32),
        pltpu.VMEM((CH, C), jnp.float32),
        pltpu.VMEM((CH, C), jnp.float32),
        pltpu.MemorySpace.VMEM_SHARED((NP, C), jnp.float32),
        pltpu.SemaphoreType.DMA,
        pltpu.SemaphoreType.DMA,
        pltpu.SemaphoreType.DMA,
        pltpu.SemaphoreType.DMA,
        pltpu.SemaphoreType.DMA,
        pltpu.SemaphoreType.DMA,
    ],
)
def _sc_conv(g_hbm, edges_hbm, zeros_hbm, out_hbm,
             src_v, dst_v, rows_a, rows_b, rows_c, rows_d, rows_e, acc_sh,
             gsem_a, gsem_b, gsem_c, gsem_d, gsem_e, zsem):
    c = lax.axis_index("c")
    s = lax.axis_index("s")
    wid = s * NC + c
    zero_cp = pltpu.async_copy(zeros_hbm, acc_sh.at[pl.ds(s * RPT, RPT)],
                               zsem)

    def gather(j, buf, sem):
        return pltpu.async_copy(g_hbm.at[src_v.at[j]], buf, sem)

    def wait_gather(j, buf, sem):
        pltpu.make_async_copy(g_hbm.at[src_v.at[j]], buf, sem).wait()

    def scatter(j, buf):
        pltpu.sync_copy(buf, acc_sh.at[dst_v.at[j]], add=True)

    bufs = [(rows_a, gsem_a), (rows_b, gsem_b), (rows_c, gsem_c),
            (rows_d, gsem_d), (rows_e, gsem_e)]
    for p in range(NPH):
        pltpu.sync_copy(edges_hbm.at[0, wid, p], src_v)
        pltpu.sync_copy(edges_hbm.at[1, wid, p], dst_v)
        for k, (buf, sem) in enumerate(bufs):
            gather(k, buf, sem)
        if p == 0:
            zero_cp.wait()
            plsc.subcore_barrier()

        def quad(j4, _):
            for k, (buf, sem) in enumerate(bufs):
                wait_gather(j4 + k, buf, sem)
                scatter(j4 + k, buf)

                @pl.when(j4 + k + NBUF < PCH)
                def _():
                    gather(j4 + k + NBUF, buf, sem)
            return 0

        lax.fori_loop(0, QUAD, lambda t, u: quad(t * NBUF, u), 0)
    plsc.subcore_barrier()
    pltpu.sync_copy(acc_sh.at[pl.ds(s * RPT, RPT)],
                    out_hbm.at[c, pl.ds(s * RPT, RPT)])


def _tc_pre_body(deg_ref, x_ref, dinv_ref, g_ref):
    deg = deg_ref[...][:, 0:1] + deg_ref[...][:, 1:2]
    dinv = jnp.where(deg > 0.0, lax.rsqrt(jnp.maximum(deg, 1e-12)), 0.0)
    dinv_ref[...] = dinv
    g_ref[...] = jnp.broadcast_to(dinv, (deg.shape[0], C)) * x_ref[...]


_RB = 1000

_tc_pre = pl.pallas_call(
    _tc_pre_body,
    grid=(N // _RB,),
    in_specs=[
        pl.BlockSpec((_RB, 2), lambda i: (i, 0)),
        pl.BlockSpec((_RB, C), lambda i: (i, 0)),
    ],
    out_specs=[
        pl.BlockSpec((_RB, 1), lambda i: (i, 0)),
        pl.BlockSpec((_RB, C), lambda i: (i, 0)),
    ],
    out_shape=[
        jax.ShapeDtypeStruct((N, 1), jnp.float32),
        jax.ShapeDtypeStruct((N, C), jnp.float32),
    ],
)


def _tc_layer_body(h_ref, a0_ref, a1_ref, dinv_ref, w0_ref, w1_ref, b_ref,
                   hn_ref, gn_ref):
    dinvb = jnp.broadcast_to(dinv_ref[...], (_RB, C))
    m = dinvb * (a0_ref[...] + a1_ref[...])
    out = (jnp.dot(h_ref[...], w0_ref[...], preferred_element_type=jnp.float32)
           - jnp.dot(m, w1_ref[...], preferred_element_type=jnp.float32)
           + b_ref[0:1, :])
    hn = jnp.maximum(out, 0.0)
    hn_ref[...] = hn
    gn_ref[...] = dinvb * hn


_tc_layer = pl.pallas_call(
    _tc_layer_body,
    grid=(N // _RB,),
    in_specs=[
        pl.BlockSpec((_RB, C), lambda i: (i, 0)),
        pl.BlockSpec((_RB, C), lambda i: (i, 0)),
        pl.BlockSpec((_RB, C), lambda i: (i, 0)),
        pl.BlockSpec((_RB, 1), lambda i: (i, 0)),
        pl.BlockSpec((C, C), lambda i: (0, 0)),
        pl.BlockSpec((C, C), lambda i: (0, 0)),
        pl.BlockSpec((8, C), lambda i: (0, 0)),
    ],
    out_specs=[
        pl.BlockSpec((_RB, C), lambda i: (i, 0)),
        pl.BlockSpec((_RB, C), lambda i: (i, 0)),
    ],
    out_shape=[
        jax.ShapeDtypeStruct((N, C), jnp.float32),
        jax.ShapeDtypeStruct((N, C), jnp.float32),
    ],
)


def _tc_last_body(h_ref, a0_ref, a1_ref, dinv_ref, w0_ref, w1_ref, b_ref,
                  batch_ref, lw1_ref, lb1_ref, lw2_ref, lb2_ref,
                  out_ref, pooled_acc, count_acc):
    i = pl.program_id(0)

    @pl.when(i == 0)
    def _():
        pooled_acc[...] = jnp.zeros_like(pooled_acc)
        count_acc[...] = jnp.zeros_like(count_acc)

    m = jnp.broadcast_to(dinv_ref[...], (_RB, C)) * (a0_ref[...] + a1_ref[...])
    hn = jnp.maximum(
        jnp.dot(h_ref[...], w0_ref[...], preferred_element_type=jnp.float32)
        - jnp.dot(m, w1_ref[...], preferred_element_type=jnp.float32)
        + b_ref[0:1, :], 0.0)

    b = batch_ref[0]
    seg = jax.lax.broadcasted_iota(jnp.int32, (NUM_GRAPHS, _RB), 0)
    onehot = (seg == jnp.broadcast_to(b, (NUM_GRAPHS, _RB))).astype(jnp.float32)
    pooled_acc[...] += jnp.dot(onehot, hn, preferred_element_type=jnp.float32)
    count_acc[...] += jnp.broadcast_to(
        jnp.sum(onehot, axis=1, keepdims=True), (NUM_GRAPHS, C))

    @pl.when(i == (N // _RB) - 1)
    def _():
        pooled = pooled_acc[...] / jnp.maximum(count_acc[...], 1.0)
        h2 = jnp.maximum(
            jnp.dot(pooled, lw1_ref[...], preferred_element_type=jnp.float32)
            + lb1_ref[0:1, :], 0.0)
        out_ref[...] = (jnp.dot(h2, lw2_ref[...],
                                preferred_element_type=jnp.float32)
                        + lb2_ref[0:1, :])


_tc_last = pl.pallas_call(
    _tc_last_body,
    grid=(N // _RB,),
    in_specs=[
        pl.BlockSpec((_RB, C), lambda i: (i, 0)),
        pl.BlockSpec((_RB, C), lambda i: (i, 0)),
        pl.BlockSpec((_RB, C), lambda i: (i, 0)),
        pl.BlockSpec((_RB, 1), lambda i: (i, 0)),
        pl.BlockSpec((C, C), lambda i: (0, 0)),
        pl.BlockSpec((C, C), lambda i: (0, 0)),
        pl.BlockSpec((8, C), lambda i: (0, 0)),
        pl.BlockSpec((1, 1, _RB), lambda i: (i, 0, 0)),
        pl.BlockSpec((C, H1), lambda i: (0, 0)),
        pl.BlockSpec((8, H1), lambda i: (0, 0)),
        pl.BlockSpec((H1, C), lambda i: (0, 0)),
        pl.BlockSpec((8, C), lambda i: (0, 0)),
    ],
    out_specs=pl.BlockSpec((NUM_GRAPHS, C), lambda i: (0, 0)),
    out_shape=jax.ShapeDtypeStruct((NUM_GRAPHS, C), jnp.float32),
    scratch_shapes=[
        pltpu.VMEM((NUM_GRAPHS, C), jnp.float32),
        pltpu.VMEM((NUM_GRAPHS, C), jnp.float32),
    ],
)


def kernel(x, edge_index, batch, conv_W, conv_b, lin1_W, lin1_b, lin2_W,
           lin2_b):
    edges_deg = edge_index.reshape(2, NW, NCHD, CHD)
    edges_conv = edge_index.reshape(2, NW, NPH, PCH, CH)
    zeros_rows = jnp.zeros((RPT, C), jnp.float32)

    deg2 = _sc_degree(edges_deg)
    degT = jnp.transpose(deg2[:, :N])
    dinv, g = _tc_pre(degT, x)

    h = x
    for i in range(NUM_CONV - 1):
        bb = jnp.broadcast_to(conv_b[i], (8, C))
        acc = _sc_conv(g, edges_conv, zeros_rows)
        h, g = _tc_layer(h, acc[0], acc[1], dinv,
                         conv_W[i, 0], conv_W[i, 1], bb)

    batch3 = batch.astype(jnp.int32).reshape(N // _RB, 1, _RB)
    b1b = jnp.broadcast_to(lin1_b, (8, H1))
    w2p = jnp.zeros((H1, C), jnp.float32).at[:, :NUM_CLASSES].set(lin2_W)
    b2p = jnp.broadcast_to(
        jnp.zeros((C,), jnp.float32).at[:NUM_CLASSES].set(lin2_b), (8, C))
    acc = _sc_conv(g, edges_conv, zeros_rows)
    bb = jnp.broadcast_to(conv_b[NUM_CONV - 1], (8, C))
    out = _tc_last(h, acc[0], acc[1], dinv,
                   conv_W[NUM_CONV - 1, 0], conv_W[NUM_CONV - 1, 1], bb,
                   batch3, lin1_W, b1b, w2p, b2p)
    return out[:, :NUM_CLASSES]
